# baseline (speedup 1.0000x reference)
import functools

import jax
import jax.numpy as jnp
from jax import lax
from jax.experimental import pallas as pl
from jax.experimental.pallas import tpu as pltpu
from jax.experimental.pallas import tpu_sc as plsc

N = 10000
E = 320000
D_IN = 128
D_H = 128
D_OUT = 40
D2P = 48

NC = 2
NS = 16
NW = NC * NS
CK = 64
CH = 158
EP = NW * CH * CK
NP = 10240
RPT = NP // NS
DEG_W = 8

def _zero_vmem(ref, rows, width):
  def zr(r, carry):
    def zc(c, carry2):
      ref[r, pl.ds(c * 16, 16)] = jnp.zeros((16,), jnp.float32)
      return carry2

    return lax.fori_loop(0, width // 16, zc, carry)

  lax.fori_loop(0, rows, zr, 0)


@functools.lru_cache(maxsize=None)
def _make_sc_degree():
  mesh = plsc.VectorSubcoreMesh(core_axis_name="c", subcore_axis_name="s")

  @functools.partial(
      pl.kernel,
      out_type=jax.ShapeDtypeStruct((NC, NP, DEG_W), jnp.float32),
      mesh=mesh,
      compiler_params=pltpu.CompilerParams(use_tc_tiling_on_sc=False),
      scratch_types=[
          pltpu.VMEM((CH, CK), jnp.int32),
          pltpu.VMEM((2, CK, DEG_W), jnp.float32),
          pltpu.VMEM_SHARED((NP, DEG_W), jnp.float32),
      ],
  )
  def _sc_degree(dstw_hbm, cz_hbm, out_hbm, dst_v, rows_v, acc):
    cid = lax.axis_index("c")
    sid = lax.axis_index("s")
    wid = sid * NC + cid
    base = sid * RPT

    pltpu.sync_copy(cz_hbm, rows_v)
    for b in range(RPT // CK):
      pltpu.sync_copy(rows_v.at[0], acc.at[pl.ds(base + b * CK, CK)])

    pltpu.sync_copy(dstw_hbm.at[wid], dst_v)
    plsc.subcore_barrier()

    def body(j, carry):
      pltpu.sync_copy(rows_v.at[1], acc.at[dst_v.at[j]], add=True)
      return carry

    lax.fori_loop(0, CH, body, 0)
    plsc.subcore_barrier()
    pltpu.sync_copy(
        acc.at[pl.ds(base, RPT)], out_hbm.at[cid, pl.ds(base, RPT)]
    )

  return _sc_degree


@functools.lru_cache(maxsize=None)
def _make_sc_scatter(D):
  mesh = plsc.VectorSubcoreMesh(core_axis_name="c", subcore_axis_name="s")

  @functools.partial(
      pl.kernel,
      out_type=jax.ShapeDtypeStruct((NC, NP, D), jnp.float32),
      mesh=mesh,
      compiler_params=pltpu.CompilerParams(use_tc_tiling_on_sc=False),
      scratch_types=[
          pltpu.VMEM((CH, CK), jnp.int32),
          pltpu.VMEM((CH, CK), jnp.int32),
          pltpu.VMEM((2, CK, D), jnp.float32),
          pltpu.VMEM_SHARED((NP, D), jnp.float32),
          pltpu.SemaphoreType.DMA,
          pltpu.SemaphoreType.DMA,
      ],
  )
  def k(
      table_hbm, srcw_hbm, dstw_hbm, out_hbm, src_v, dst_v, rows_v, acc,
      sem_g, sem_s,
  ):
    cid = lax.axis_index("c")
    sid = lax.axis_index("s")
    wid = sid * NC + cid
    base = sid * RPT

    _zero_vmem(rows_v.at[0], CK, D)
    for b in range(RPT // CK):
      pltpu.sync_copy(rows_v.at[0], acc.at[pl.ds(base + b * CK, CK)])

    pltpu.sync_copy(srcw_hbm.at[wid], src_v)
    pltpu.sync_copy(dstw_hbm.at[wid], dst_v)
    plsc.subcore_barrier()

    pltpu.async_copy(table_hbm.at[src_v.at[0]], rows_v.at[0], sem_g)

    def body(j, carry):
      b = lax.rem(j, 2)
      pltpu.make_async_copy(
          table_hbm.at[src_v.at[j]], rows_v.at[b], sem_g
      ).wait()

      @pl.when(j > 0)
      def _():
        pltpu.make_async_copy(
            rows_v.at[1 - b], acc.at[dst_v.at[j - 1]], sem_s
        ).wait()

      @pl.when(j + 1 < CH)
      def _():
        pltpu.async_copy(table_hbm.at[src_v.at[j + 1]], rows_v.at[1 - b], sem_g)

      pltpu.async_copy(rows_v.at[b], acc.at[dst_v.at[j]], sem_s, add=True)
      return carry

    lax.fori_loop(0, CH, body, 0)
    pltpu.make_async_copy(
        rows_v.at[(CH - 1) % 2], acc.at[dst_v.at[CH - 1]], sem_s
    ).wait()
    plsc.subcore_barrier()
    pltpu.sync_copy(
        acc.at[pl.ds(base, RPT)], out_hbm.at[cid, pl.ds(base, RPT)]
    )

  return k


_RB = 1000
_GRID = N // _RB


def _dinv_block(d_ref):
  deg = d_ref[0, :, :1] + d_ref[1, :, :1] + 1.0
  return lax.rsqrt(deg)


def _mm1_body(x_ref, w_ref, o_ref):
  o_ref[...] = jnp.dot(
      x_ref[...], w_ref[...], preferred_element_type=jnp.float32
  )


def _scale_body(h_ref, d_ref, o_ref):
  o_ref[...] = h_ref[...] * _dinv_block(d_ref)


def _mid_body(p_ref, g1_ref, d_ref, b1_ref, w2_ref, o_ref):
  dinv = _dinv_block(d_ref)
  s = p_ref[0] + p_ref[1] + g1_ref[...]
  t = jnp.maximum(s * dinv + b1_ref[...], 0.0)
  o_ref[...] = (
      jnp.dot(t, w2_ref[...], preferred_element_type=jnp.float32) * dinv
  )


def _final_body(p_ref, g2_ref, d_ref, b2_ref, o_ref):
  dinv = _dinv_block(d_ref)
  s = p_ref[0] + p_ref[1] + g2_ref[...]
  o = s[:, :D_OUT] * dinv + b2_ref[...]
  m = jnp.max(o, axis=1, keepdims=True)
  lse = jnp.log(jnp.sum(jnp.exp(o - m), axis=1, keepdims=True)) + m
  o_ref[...] = o - lse


def _row_spec(w):
  return pl.BlockSpec((_RB, w), lambda i: (i, 0))


def _full_spec(shape):
  nd = len(shape)
  return pl.BlockSpec(shape, lambda i: (0,) * nd)


def _deg_spec():
  return pl.BlockSpec((NC, _RB, DEG_W), lambda i: (0, i, 0))


def _part_spec(w):
  return pl.BlockSpec((NC, _RB, w), lambda i: (0, i, 0))


def kernel(x, edge_index, W1, b1, W2, b2):
  src = edge_index[0]
  dst = edge_index[1]
  pad = EP - E
  ar = jnp.arange(pad, dtype=jnp.int32)
  srcw = jnp.concatenate([src, ar % N]).reshape(NW, CH, CK)
  dstw = jnp.concatenate([dst, N + ar % CK]).reshape(NW, CH, CK)

  cz = jnp.stack([
      jnp.zeros((CK, DEG_W), jnp.float32),
      jnp.ones((CK, DEG_W), jnp.float32),
  ])
  degp = _make_sc_degree()(dstw, cz)

  h1 = pl.pallas_call(
      _mm1_body,
      grid=(_GRID,),
      in_specs=[_row_spec(D_IN), _full_spec((D_IN, D_H))],
      out_specs=_row_spec(D_H),
      out_shape=jax.ShapeDtypeStruct((N, D_H), jnp.float32),
  )(x, W1)

  g1 = pl.pallas_call(
      _scale_body,
      grid=(_GRID,),
      in_specs=[_row_spec(D_H), _deg_spec()],
      out_specs=_row_spec(D_H),
      out_shape=jax.ShapeDtypeStruct((N, D_H), jnp.float32),
  )(h1, degp)

  p1 = _make_sc_scatter(D_H)(g1, srcw, dstw)

  b1r = b1.reshape(1, D_H)
  w2p = jnp.zeros((D_H, D2P), jnp.float32).at[:, :D_OUT].set(W2)
  g2 = pl.pallas_call(
      _mid_body,
      grid=(_GRID,),
      in_specs=[
          _part_spec(D_H),
          _row_spec(D_H),
          _deg_spec(),
          _full_spec((1, D_H)),
          _full_spec((D_H, D2P)),
      ],
      out_specs=_row_spec(D2P),
      out_shape=jax.ShapeDtypeStruct((N, D2P), jnp.float32),
  )(p1, g1, degp, b1r, w2p)

  p2 = _make_sc_scatter(D2P)(g2, srcw, dstw)

  b2r = b2.reshape(1, D_OUT)
  out = pl.pallas_call(
      _final_body,
      grid=(_GRID,),
      in_specs=[
          _part_spec(D2P),
          _row_spec(D2P),
          _deg_spec(),
          _full_spec((1, D_OUT)),
      ],
      out_specs=_row_spec(D_OUT),
      out_shape=jax.ShapeDtypeStruct((N, D_OUT), jnp.float32),
  )(p2, g2, degp, b2r)

  return out


# speedup vs baseline: 1.1808x; 1.1808x over previous
import functools

import jax
import jax.numpy as jnp
from jax import lax
from jax.experimental import pallas as pl
from jax.experimental.pallas import tpu as pltpu
from jax.experimental.pallas import tpu_sc as plsc

N = 10000
E = 320000
D_IN = 128
D_H = 128
D_OUT = 40
D2P = 48

NC = 2
NS = 16
NW = NC * NS
CK = 96
CH = 105
EP = NW * CH * CK
NP = 10240
RPT = NP // NS
ZR = 64
DEG_W = 8

def _zero_vmem(ref, rows, width):
  def zr(r, carry):
    def zc(c, carry2):
      ref[r, pl.ds(c * 16, 16)] = jnp.zeros((16,), jnp.float32)
      return carry2

    return lax.fori_loop(0, width // 16, zc, carry)

  lax.fori_loop(0, rows, zr, 0)


@functools.lru_cache(maxsize=None)
def _make_sc_degree():
  mesh = plsc.VectorSubcoreMesh(core_axis_name="c", subcore_axis_name="s")

  @functools.partial(
      pl.kernel,
      out_type=jax.ShapeDtypeStruct((NC, NP, DEG_W), jnp.float32),
      mesh=mesh,
      compiler_params=pltpu.CompilerParams(use_tc_tiling_on_sc=False),
      scratch_types=[
          pltpu.VMEM((CH, CK), jnp.int32),
          pltpu.VMEM((2, CK, DEG_W), jnp.float32),
          pltpu.VMEM_SHARED((NP, DEG_W), jnp.float32),
      ],
  )
  def _sc_degree(dstw_hbm, cz_hbm, out_hbm, dst_v, rows_v, acc):
    cid = lax.axis_index("c")
    sid = lax.axis_index("s")
    wid = sid * NC + cid
    base = sid * RPT

    pltpu.sync_copy(cz_hbm, rows_v)
    for b in range(RPT // ZR):
      pltpu.sync_copy(
          rows_v.at[0, pl.ds(0, ZR)], acc.at[pl.ds(base + b * ZR, ZR)]
      )

    pltpu.sync_copy(dstw_hbm.at[wid], dst_v)
    plsc.subcore_barrier()

    def body(j, carry):
      pltpu.sync_copy(rows_v.at[1], acc.at[dst_v.at[j]], add=True)
      return carry

    lax.fori_loop(0, CH, body, 0)
    plsc.subcore_barrier()
    pltpu.sync_copy(
        acc.at[pl.ds(base, RPT)], out_hbm.at[cid, pl.ds(base, RPT)]
    )

  return _sc_degree


@functools.lru_cache(maxsize=None)
def _make_sc_scatter(D):
  mesh = plsc.VectorSubcoreMesh(core_axis_name="c", subcore_axis_name="s")

  @functools.partial(
      pl.kernel,
      out_type=jax.ShapeDtypeStruct((NC, NP, D), jnp.float32),
      mesh=mesh,
      compiler_params=pltpu.CompilerParams(use_tc_tiling_on_sc=False),
      scratch_types=[
          pltpu.VMEM((CH, CK), jnp.int32),
          pltpu.VMEM((CH, CK), jnp.int32),
          pltpu.VMEM((2, CK, D), jnp.float32),
          pltpu.VMEM_SHARED((NP, D), jnp.float32),
          pltpu.SemaphoreType.DMA,
          pltpu.SemaphoreType.DMA,
      ],
  )
  def k(
      table_hbm, srcw_hbm, dstw_hbm, out_hbm, src_v, dst_v, rows_v, acc,
      sem_g, sem_s,
  ):
    cid = lax.axis_index("c")
    sid = lax.axis_index("s")
    wid = sid * NC + cid
    base = sid * RPT

    _zero_vmem(rows_v.at[0], CK, D)
    for b in range(RPT // ZR):
      pltpu.sync_copy(
          rows_v.at[0, pl.ds(0, ZR)], acc.at[pl.ds(base + b * ZR, ZR)]
      )

    pltpu.sync_copy(srcw_hbm.at[wid], src_v)
    pltpu.sync_copy(dstw_hbm.at[wid], dst_v)
    plsc.subcore_barrier()

    pltpu.async_copy(table_hbm.at[src_v.at[0]], rows_v.at[0], sem_g)

    def body(j, carry):
      b = lax.rem(j, 2)
      pltpu.make_async_copy(
          table_hbm.at[src_v.at[j]], rows_v.at[b], sem_g
      ).wait()

      @pl.when(j > 0)
      def _():
        pltpu.make_async_copy(
            rows_v.at[1 - b], acc.at[dst_v.at[j - 1]], sem_s
        ).wait()

      @pl.when(j + 1 < CH)
      def _():
        pltpu.async_copy(table_hbm.at[src_v.at[j + 1]], rows_v.at[1 - b], sem_g)

      pltpu.async_copy(rows_v.at[b], acc.at[dst_v.at[j]], sem_s, add=True)
      return carry

    lax.fori_loop(0, CH, body, 0)
    pltpu.make_async_copy(
        rows_v.at[(CH - 1) % 2], acc.at[dst_v.at[CH - 1]], sem_s
    ).wait()
    plsc.subcore_barrier()
    pltpu.sync_copy(
        acc.at[pl.ds(base, RPT)], out_hbm.at[cid, pl.ds(base, RPT)]
    )

  return k


_RB = 1000
_GRID = N // _RB


def _dinv_block(d_ref):
  deg = d_ref[0, :, :1] + d_ref[1, :, :1] + 1.0
  return lax.rsqrt(deg)


def _mm1_body(x_ref, w_ref, o_ref):
  o_ref[...] = jnp.dot(
      x_ref[...], w_ref[...], preferred_element_type=jnp.float32
  )


def _scale_body(h_ref, d_ref, o_ref):
  o_ref[...] = h_ref[...] * _dinv_block(d_ref)


def _mid_body(p_ref, g1_ref, d_ref, b1_ref, w2_ref, o_ref):
  dinv = _dinv_block(d_ref)
  s = p_ref[0] + p_ref[1] + g1_ref[...]
  t = jnp.maximum(s * dinv + b1_ref[...], 0.0)
  o_ref[...] = (
      jnp.dot(t, w2_ref[...], preferred_element_type=jnp.float32) * dinv
  )


def _final_body(p_ref, g2_ref, d_ref, b2_ref, o_ref):
  dinv = _dinv_block(d_ref)
  s = p_ref[0] + p_ref[1] + g2_ref[...]
  o = s[:, :D_OUT] * dinv + b2_ref[...]
  m = jnp.max(o, axis=1, keepdims=True)
  lse = jnp.log(jnp.sum(jnp.exp(o - m), axis=1, keepdims=True)) + m
  o_ref[...] = o - lse


def _row_spec(w):
  return pl.BlockSpec((_RB, w), lambda i: (i, 0))


def _full_spec(shape):
  nd = len(shape)
  return pl.BlockSpec(shape, lambda i: (0,) * nd)


def _deg_spec():
  return pl.BlockSpec((NC, _RB, DEG_W), lambda i: (0, i, 0))


def _part_spec(w):
  return pl.BlockSpec((NC, _RB, w), lambda i: (0, i, 0))


def kernel(x, edge_index, W1, b1, W2, b2):
  src = edge_index[0]
  dst = edge_index[1]
  pad = EP - E
  ar = jnp.arange(pad, dtype=jnp.int32)
  srcw = jnp.concatenate([src, ar % N]).reshape(NW, CH, CK)
  dstw = jnp.concatenate([dst, N + ar % CK]).reshape(NW, CH, CK)

  cz = jnp.stack([
      jnp.zeros((CK, DEG_W), jnp.float32),
      jnp.ones((CK, DEG_W), jnp.float32),
  ])
  degp = _make_sc_degree()(dstw, cz)

  h1 = pl.pallas_call(
      _mm1_body,
      grid=(_GRID,),
      in_specs=[_row_spec(D_IN), _full_spec((D_IN, D_H))],
      out_specs=_row_spec(D_H),
      out_shape=jax.ShapeDtypeStruct((N, D_H), jnp.float32),
  )(x, W1)

  g1 = pl.pallas_call(
      _scale_body,
      grid=(_GRID,),
      in_specs=[_row_spec(D_H), _deg_spec()],
      out_specs=_row_spec(D_H),
      out_shape=jax.ShapeDtypeStruct((N, D_H), jnp.float32),
  )(h1, degp)

  p1 = _make_sc_scatter(D_H)(g1, srcw, dstw)

  b1r = b1.reshape(1, D_H)
  w2p = jnp.zeros((D_H, D2P), jnp.float32).at[:, :D_OUT].set(W2)
  g2 = pl.pallas_call(
      _mid_body,
      grid=(_GRID,),
      in_specs=[
          _part_spec(D_H),
          _row_spec(D_H),
          _deg_spec(),
          _full_spec((1, D_H)),
          _full_spec((D_H, D2P)),
      ],
      out_specs=_row_spec(D2P),
      out_shape=jax.ShapeDtypeStruct((N, D2P), jnp.float32),
  )(p1, g1, degp, b1r, w2p)

  p2 = _make_sc_scatter(D2P)(g2, srcw, dstw)

  b2r = b2.reshape(1, D_OUT)
  out = pl.pallas_call(
      _final_body,
      grid=(_GRID,),
      in_specs=[
          _part_spec(D2P),
          _row_spec(D2P),
          _deg_spec(),
          _full_spec((1, D_OUT)),
      ],
      out_specs=_row_spec(D_OUT),
      out_shape=jax.ShapeDtypeStruct((N, D_OUT), jnp.float32),
  )(p2, g2, degp, b2r)

  return out


# speedup vs baseline: 1.2433x; 1.0529x over previous
import functools

import jax
import jax.numpy as jnp
from jax import lax
from jax.experimental import pallas as pl
from jax.experimental.pallas import tpu as pltpu
from jax.experimental.pallas import tpu_sc as plsc

N = 10000
E = 320000
D_IN = 128
D_H = 128
D_OUT = 40
D2P = 48

NC = 2
NS = 16
NW = NC * NS
CK = 112
CH = 90
EP = NW * CH * CK
NP = 10240
RPT = NP // NS
ZR = 64
DEG_W = 8

def _zero_vmem(ref, rows, width):
  def zr(r, carry):
    def zc(c, carry2):
      ref[r, pl.ds(c * 16, 16)] = jnp.zeros((16,), jnp.float32)
      return carry2

    return lax.fori_loop(0, width // 16, zc, carry)

  lax.fori_loop(0, rows, zr, 0)


@functools.lru_cache(maxsize=None)
def _make_sc_degree():
  mesh = plsc.VectorSubcoreMesh(core_axis_name="c", subcore_axis_name="s")

  @functools.partial(
      pl.kernel,
      out_type=jax.ShapeDtypeStruct((NC, NP, DEG_W), jnp.float32),
      mesh=mesh,
      compiler_params=pltpu.CompilerParams(use_tc_tiling_on_sc=False),
      scratch_types=[
          pltpu.VMEM((CH, CK), jnp.int32),
          pltpu.VMEM((2, CK, DEG_W), jnp.float32),
          pltpu.VMEM_SHARED((NP, DEG_W), jnp.float32),
      ],
  )
  def _sc_degree(dstw_hbm, cz_hbm, out_hbm, dst_v, rows_v, acc):
    cid = lax.axis_index("c")
    sid = lax.axis_index("s")
    wid = sid * NC + cid
    base = sid * RPT

    pltpu.sync_copy(cz_hbm, rows_v)
    for b in range(RPT // ZR):
      pltpu.sync_copy(
          rows_v.at[0, pl.ds(0, ZR)], acc.at[pl.ds(base + b * ZR, ZR)]
      )

    pltpu.sync_copy(dstw_hbm.at[wid], dst_v)
    plsc.subcore_barrier()

    def body(j, carry):
      pltpu.sync_copy(rows_v.at[1], acc.at[dst_v.at[j]], add=True)
      return carry

    lax.fori_loop(0, CH, body, 0)
    plsc.subcore_barrier()
    pltpu.sync_copy(
        acc.at[pl.ds(base, RPT)], out_hbm.at[cid, pl.ds(base, RPT)]
    )

  return _sc_degree


@functools.lru_cache(maxsize=None)
def _make_sc_scatter(D):
  mesh = plsc.VectorSubcoreMesh(core_axis_name="c", subcore_axis_name="s")

  @functools.partial(
      pl.kernel,
      out_type=jax.ShapeDtypeStruct((NC, NP, D), jnp.float32),
      mesh=mesh,
      compiler_params=pltpu.CompilerParams(use_tc_tiling_on_sc=False),
      scratch_types=[
          pltpu.VMEM((CH, CK), jnp.int32),
          pltpu.VMEM((CH, CK), jnp.int32),
          pltpu.VMEM((2, CK, D), jnp.float32),
          pltpu.VMEM_SHARED((NP, D), jnp.float32),
          pltpu.SemaphoreType.DMA,
          pltpu.SemaphoreType.DMA,
      ],
  )
  def k(
      table_hbm, srcw_hbm, dstw_hbm, out_hbm, src_v, dst_v, rows_v, acc,
      sem_g, sem_s,
  ):
    cid = lax.axis_index("c")
    sid = lax.axis_index("s")
    wid = sid * NC + cid
    base = sid * RPT

    _zero_vmem(rows_v.at[0], CK, D)
    for b in range(RPT // ZR):
      pltpu.sync_copy(
          rows_v.at[0, pl.ds(0, ZR)], acc.at[pl.ds(base + b * ZR, ZR)]
      )

    pltpu.sync_copy(srcw_hbm.at[wid], src_v)
    pltpu.sync_copy(dstw_hbm.at[wid], dst_v)
    plsc.subcore_barrier()

    pltpu.async_copy(table_hbm.at[src_v.at[0]], rows_v.at[0], sem_g)

    def body(j, carry):
      b = lax.rem(j, 2)
      pltpu.make_async_copy(
          table_hbm.at[src_v.at[j]], rows_v.at[b], sem_g
      ).wait()

      @pl.when(j > 0)
      def _():
        pltpu.make_async_copy(
            rows_v.at[1 - b], acc.at[dst_v.at[j - 1]], sem_s
        ).wait()

      @pl.when(j + 1 < CH)
      def _():
        pltpu.async_copy(table_hbm.at[src_v.at[j + 1]], rows_v.at[1 - b], sem_g)

      pltpu.async_copy(rows_v.at[b], acc.at[dst_v.at[j]], sem_s, add=True)
      return carry

    lax.fori_loop(0, CH, body, 0)
    pltpu.make_async_copy(
        rows_v.at[(CH - 1) % 2], acc.at[dst_v.at[CH - 1]], sem_s
    ).wait()
    plsc.subcore_barrier()
    pltpu.sync_copy(
        acc.at[pl.ds(base, RPT)], out_hbm.at[cid, pl.ds(base, RPT)]
    )

  return k


_RB = 1000
_GRID = N // _RB


def _dinv_block(d_ref):
  deg = d_ref[0, :, :1] + d_ref[1, :, :1] + 1.0
  return lax.rsqrt(deg)


def _mm1_body(x_ref, w_ref, o_ref):
  o_ref[...] = jnp.dot(
      x_ref[...], w_ref[...], preferred_element_type=jnp.float32
  )


def _scale_body(h_ref, d_ref, o_ref):
  o_ref[...] = h_ref[...] * _dinv_block(d_ref)


def _mid_body(p_ref, g1_ref, d_ref, b1_ref, w2_ref, o_ref):
  dinv = _dinv_block(d_ref)
  s = p_ref[0] + p_ref[1] + g1_ref[...]
  t = jnp.maximum(s * dinv + b1_ref[...], 0.0)
  o_ref[...] = (
      jnp.dot(t, w2_ref[...], preferred_element_type=jnp.float32) * dinv
  )


def _final_body(p_ref, g2_ref, d_ref, b2_ref, o_ref):
  dinv = _dinv_block(d_ref)
  s = p_ref[0] + p_ref[1] + g2_ref[...]
  o = s[:, :D_OUT] * dinv + b2_ref[...]
  m = jnp.max(o, axis=1, keepdims=True)
  lse = jnp.log(jnp.sum(jnp.exp(o - m), axis=1, keepdims=True)) + m
  o_ref[...] = o - lse


def _row_spec(w):
  return pl.BlockSpec((_RB, w), lambda i: (i, 0))


def _full_spec(shape):
  nd = len(shape)
  return pl.BlockSpec(shape, lambda i: (0,) * nd)


def _deg_spec():
  return pl.BlockSpec((NC, _RB, DEG_W), lambda i: (0, i, 0))


def _part_spec(w):
  return pl.BlockSpec((NC, _RB, w), lambda i: (0, i, 0))


def kernel(x, edge_index, W1, b1, W2, b2):
  src = edge_index[0]
  dst = edge_index[1]
  pad = EP - E
  ar = jnp.arange(pad, dtype=jnp.int32)
  srcw = jnp.concatenate([src, ar % N]).reshape(NW, CH, CK)
  dstw = jnp.concatenate([dst, N + ar % CK]).reshape(NW, CH, CK)

  cz = jnp.stack([
      jnp.zeros((CK, DEG_W), jnp.float32),
      jnp.ones((CK, DEG_W), jnp.float32),
  ])
  degp = _make_sc_degree()(dstw, cz)

  h1 = pl.pallas_call(
      _mm1_body,
      grid=(_GRID,),
      in_specs=[_row_spec(D_IN), _full_spec((D_IN, D_H))],
      out_specs=_row_spec(D_H),
      out_shape=jax.ShapeDtypeStruct((N, D_H), jnp.float32),
  )(x, W1)

  g1 = pl.pallas_call(
      _scale_body,
      grid=(_GRID,),
      in_specs=[_row_spec(D_H), _deg_spec()],
      out_specs=_row_spec(D_H),
      out_shape=jax.ShapeDtypeStruct((N, D_H), jnp.float32),
  )(h1, degp)

  p1 = _make_sc_scatter(D_H)(g1, srcw, dstw)

  b1r = b1.reshape(1, D_H)
  w2p = jnp.zeros((D_H, D2P), jnp.float32).at[:, :D_OUT].set(W2)
  g2 = pl.pallas_call(
      _mid_body,
      grid=(_GRID,),
      in_specs=[
          _part_spec(D_H),
          _row_spec(D_H),
          _deg_spec(),
          _full_spec((1, D_H)),
          _full_spec((D_H, D2P)),
      ],
      out_specs=_row_spec(D2P),
      out_shape=jax.ShapeDtypeStruct((N, D2P), jnp.float32),
  )(p1, g1, degp, b1r, w2p)

  p2 = _make_sc_scatter(D2P)(g2, srcw, dstw)

  b2r = b2.reshape(1, D_OUT)
  out = pl.pallas_call(
      _final_body,
      grid=(_GRID,),
      in_specs=[
          _part_spec(D2P),
          _row_spec(D2P),
          _deg_spec(),
          _full_spec((1, D_OUT)),
      ],
      out_specs=_row_spec(D_OUT),
      out_shape=jax.ShapeDtypeStruct((N, D_OUT), jnp.float32),
  )(p2, g2, degp, b2r)

  return out


# speedup vs baseline: 1.2511x; 1.0063x over previous
import functools

import jax
import jax.numpy as jnp
from jax import lax
from jax.experimental import pallas as pl
from jax.experimental.pallas import tpu as pltpu
from jax.experimental.pallas import tpu_sc as plsc

N = 10000
E = 320000
D_IN = 128
D_H = 128
D_OUT = 40
D2P = 48

NC = 2
NS = 16
NW = NC * NS
CK = 112
CH = 90
EP = NW * CH * CK
NP = 10240
RPT = NP // NS
ZR = 64
DEG_W = 8

def _zero_vmem(ref, rows, width):
  def zr(r, carry):
    def zc(c, carry2):
      ref[r, pl.ds(c * 16, 16)] = jnp.zeros((16,), jnp.float32)
      return carry2

    return lax.fori_loop(0, width // 16, zc, carry)

  lax.fori_loop(0, rows, zr, 0)


@functools.lru_cache(maxsize=None)
def _make_sc_degree():
  mesh = plsc.VectorSubcoreMesh(core_axis_name="c", subcore_axis_name="s")

  @functools.partial(
      pl.kernel,
      out_type=jax.ShapeDtypeStruct((NC, NP, DEG_W), jnp.float32),
      mesh=mesh,
      compiler_params=pltpu.CompilerParams(use_tc_tiling_on_sc=False),
      scratch_types=[
          pltpu.VMEM((CH, CK), jnp.int32),
          pltpu.VMEM((2, CK, DEG_W), jnp.float32),
          pltpu.VMEM_SHARED((NP, DEG_W), jnp.float32),
      ],
  )
  def _sc_degree(dstw_hbm, cz_hbm, out_hbm, dst_v, rows_v, acc):
    cid = lax.axis_index("c")
    sid = lax.axis_index("s")
    wid = sid * NC + cid
    base = sid * RPT

    pltpu.sync_copy(cz_hbm, rows_v)
    for b in range(RPT // ZR):
      pltpu.sync_copy(
          rows_v.at[0, pl.ds(0, ZR)], acc.at[pl.ds(base + b * ZR, ZR)]
      )

    pltpu.sync_copy(dstw_hbm.at[wid], dst_v)
    plsc.subcore_barrier()

    def body(j, carry):
      pltpu.sync_copy(rows_v.at[1], acc.at[dst_v.at[j]], add=True)
      return carry

    lax.fori_loop(0, CH, body, 0)
    plsc.subcore_barrier()
    pltpu.sync_copy(
        acc.at[pl.ds(base, RPT)], out_hbm.at[cid, pl.ds(base, RPT)]
    )

  return _sc_degree


@functools.lru_cache(maxsize=None)
def _make_sc_scatter(D):
  mesh = plsc.VectorSubcoreMesh(core_axis_name="c", subcore_axis_name="s")

  @functools.partial(
      pl.kernel,
      out_type=jax.ShapeDtypeStruct((NC, NP, D), jnp.float32),
      mesh=mesh,
      compiler_params=pltpu.CompilerParams(use_tc_tiling_on_sc=False),
      scratch_types=[
          pltpu.VMEM((CH, CK), jnp.int32),
          pltpu.VMEM((CH, CK), jnp.int32),
          pltpu.VMEM((2, CK, D), jnp.float32),
          pltpu.VMEM_SHARED((NP, D), jnp.float32),
          pltpu.SemaphoreType.DMA,
          pltpu.SemaphoreType.DMA,
      ],
  )
  def k(
      table_hbm, srcw_hbm, dstw_hbm, out_hbm, src_v, dst_v, rows_v, acc,
      sem_g, sem_s,
  ):
    cid = lax.axis_index("c")
    sid = lax.axis_index("s")
    wid = sid * NC + cid
    base = sid * RPT

    _zero_vmem(rows_v.at[0], CK, D)
    for b in range(RPT // ZR):
      pltpu.sync_copy(
          rows_v.at[0, pl.ds(0, ZR)], acc.at[pl.ds(base + b * ZR, ZR)]
      )

    pltpu.sync_copy(srcw_hbm.at[wid], src_v)
    pltpu.sync_copy(dstw_hbm.at[wid], dst_v)
    plsc.subcore_barrier()

    pltpu.async_copy(table_hbm.at[src_v.at[0]], rows_v.at[0], sem_g)

    def body(j, carry):
      b = lax.rem(j, 2)
      pltpu.make_async_copy(
          table_hbm.at[src_v.at[j]], rows_v.at[b], sem_g
      ).wait()

      @pl.when(j > 0)
      def _():
        pltpu.make_async_copy(
            rows_v.at[1 - b], acc.at[dst_v.at[j - 1]], sem_s
        ).wait()

      @pl.when(j + 1 < CH)
      def _():
        pltpu.async_copy(table_hbm.at[src_v.at[j + 1]], rows_v.at[1 - b], sem_g)

      pltpu.async_copy(rows_v.at[b], acc.at[dst_v.at[j]], sem_s, add=True)
      return carry

    lax.fori_loop(0, CH, body, 0)
    pltpu.make_async_copy(
        rows_v.at[(CH - 1) % 2], acc.at[dst_v.at[CH - 1]], sem_s
    ).wait()
    plsc.subcore_barrier()
    pltpu.sync_copy(
        acc.at[pl.ds(base, RPT)], out_hbm.at[cid, pl.ds(base, RPT)]
    )

  return k


_RB = 1000
_GRID = N // _RB


def _dinv_block(d_ref):
  deg = d_ref[0, :, :1] + d_ref[1, :, :1] + 1.0
  return lax.rsqrt(deg)


def _mm1_body(x_ref, w_ref, d_ref, o_ref):
  o_ref[...] = jnp.dot(
      x_ref[...], w_ref[...], preferred_element_type=jnp.float32
  ) * _dinv_block(d_ref)


def _mid_body(p_ref, g1_ref, d_ref, b1_ref, w2_ref, o_ref):
  dinv = _dinv_block(d_ref)
  s = p_ref[0] + p_ref[1] + g1_ref[...]
  t = jnp.maximum(s * dinv + b1_ref[...], 0.0)
  o_ref[...] = (
      jnp.dot(t, w2_ref[...], preferred_element_type=jnp.float32) * dinv
  )


def _final_body(p_ref, g2_ref, d_ref, b2_ref, o_ref):
  dinv = _dinv_block(d_ref)
  s = p_ref[0] + p_ref[1] + g2_ref[...]
  o = s[:, :D_OUT] * dinv + b2_ref[...]
  m = jnp.max(o, axis=1, keepdims=True)
  lse = jnp.log(jnp.sum(jnp.exp(o - m), axis=1, keepdims=True)) + m
  o_ref[...] = o - lse


def _row_spec(w):
  return pl.BlockSpec((_RB, w), lambda i: (i, 0))


def _full_spec(shape):
  nd = len(shape)
  return pl.BlockSpec(shape, lambda i: (0,) * nd)


def _deg_spec():
  return pl.BlockSpec((NC, _RB, DEG_W), lambda i: (0, i, 0))


def _part_spec(w):
  return pl.BlockSpec((NC, _RB, w), lambda i: (0, i, 0))


def kernel(x, edge_index, W1, b1, W2, b2):
  src = edge_index[0]
  dst = edge_index[1]
  pad = EP - E
  ar = jnp.arange(pad, dtype=jnp.int32)
  srcw = jnp.concatenate([src, ar % N]).reshape(NW, CH, CK)
  dstw = jnp.concatenate([dst, N + ar % CK]).reshape(NW, CH, CK)

  cz = jnp.stack([
      jnp.zeros((CK, DEG_W), jnp.float32),
      jnp.ones((CK, DEG_W), jnp.float32),
  ])
  degp = _make_sc_degree()(dstw, cz)

  g1 = pl.pallas_call(
      _mm1_body,
      grid=(_GRID,),
      in_specs=[_row_spec(D_IN), _full_spec((D_IN, D_H)), _deg_spec()],
      out_specs=_row_spec(D_H),
      out_shape=jax.ShapeDtypeStruct((N, D_H), jnp.float32),
  )(x, W1, degp)

  p1 = _make_sc_scatter(D_H)(g1, srcw, dstw)

  b1r = b1.reshape(1, D_H)
  w2p = jnp.zeros((D_H, D2P), jnp.float32).at[:, :D_OUT].set(W2)
  g2 = pl.pallas_call(
      _mid_body,
      grid=(_GRID,),
      in_specs=[
          _part_spec(D_H),
          _row_spec(D_H),
          _deg_spec(),
          _full_spec((1, D_H)),
          _full_spec((D_H, D2P)),
      ],
      out_specs=_row_spec(D2P),
      out_shape=jax.ShapeDtypeStruct((N, D2P), jnp.float32),
  )(p1, g1, degp, b1r, w2p)

  p2 = _make_sc_scatter(D2P)(g2, srcw, dstw)

  b2r = b2.reshape(1, D_OUT)
  out = pl.pallas_call(
      _final_body,
      grid=(_GRID,),
      in_specs=[
          _part_spec(D2P),
          _row_spec(D2P),
          _deg_spec(),
          _full_spec((1, D_OUT)),
      ],
      out_specs=_row_spec(D_OUT),
      out_shape=jax.ShapeDtypeStruct((N, D_OUT), jnp.float32),
  )(p2, g2, degp, b2r)

  return out


# speedup vs baseline: 1.2600x; 1.0071x over previous
import functools

import jax
import jax.numpy as jnp
from jax import lax
from jax.experimental import pallas as pl
from jax.experimental.pallas import tpu as pltpu
from jax.experimental.pallas import tpu_sc as plsc

N = 10000
E = 320000
D_IN = 128
D_H = 128
D_OUT = 40
D2P = 48

NC = 2
NS = 16
NW = NC * NS
CK = 112
CH = 90
EP = NW * CH * CK
NP = 10240
RPT = NP // NS
ZR = 64
IR = 8
DEG_W = 8

def _zero_vmem(ref, rows, width):
  def zr(r, carry):
    def zc(c, carry2):
      ref[r, pl.ds(c * 16, 16)] = jnp.zeros((16,), jnp.float32)
      return carry2

    return lax.fori_loop(0, width // 16, zc, carry)

  lax.fori_loop(0, rows, zr, 0)


@functools.lru_cache(maxsize=None)
def _make_sc_degree():
  mesh = plsc.VectorSubcoreMesh(core_axis_name="c", subcore_axis_name="s")

  @functools.partial(
      pl.kernel,
      out_type=jax.ShapeDtypeStruct((NC, NP, DEG_W), jnp.float32),
      mesh=mesh,
      compiler_params=pltpu.CompilerParams(use_tc_tiling_on_sc=False),
      scratch_types=[
          pltpu.VMEM((CH, CK), jnp.int32),
          pltpu.VMEM((2, CK, DEG_W), jnp.float32),
          pltpu.VMEM_SHARED((NP, DEG_W), jnp.float32),
      ],
  )
  def _sc_degree(dstw_hbm, cz_hbm, out_hbm, dst_v, rows_v, acc):
    cid = lax.axis_index("c")
    sid = lax.axis_index("s")
    wid = sid * NC + cid
    base = sid * RPT

    pltpu.sync_copy(cz_hbm, rows_v)
    for b in range(RPT // ZR):
      pltpu.sync_copy(
          rows_v.at[0, pl.ds(0, ZR)], acc.at[pl.ds(base + b * ZR, ZR)]
      )

    pltpu.sync_copy(dstw_hbm.at[wid], dst_v)
    plsc.subcore_barrier()

    def body(j, carry):
      pltpu.sync_copy(rows_v.at[1], acc.at[dst_v.at[j]], add=True)
      return carry

    lax.fori_loop(0, CH, body, 0)
    plsc.subcore_barrier()
    pltpu.sync_copy(
        acc.at[pl.ds(base, RPT)], out_hbm.at[cid, pl.ds(base, RPT)]
    )

  return _sc_degree


@functools.lru_cache(maxsize=None)
def _make_sc_scatter(D):
  mesh = plsc.VectorSubcoreMesh(core_axis_name="c", subcore_axis_name="s")

  @functools.partial(
      pl.kernel,
      out_type=jax.ShapeDtypeStruct((NC, NP, D), jnp.float32),
      mesh=mesh,
      compiler_params=pltpu.CompilerParams(use_tc_tiling_on_sc=False),
      scratch_types=[
          pltpu.VMEM((IR, CK), jnp.int32),
          pltpu.VMEM((IR, CK), jnp.int32),
          pltpu.VMEM((3, CK, D), jnp.float32),
          pltpu.VMEM_SHARED((NP, D), jnp.float32),
          pltpu.SemaphoreType.DMA,
          pltpu.SemaphoreType.DMA,
          pltpu.SemaphoreType.DMA,
          pltpu.SemaphoreType.DMA,
      ],
  )
  def k(
      table_hbm, srcw_hbm, dstw_hbm, out_hbm, src_v, dst_v, rows_v, acc,
      sem_g, sem_s, sem_is, sem_id,
  ):
    cid = lax.axis_index("c")
    sid = lax.axis_index("s")
    wid = sid * NC + cid
    base = sid * RPT

    for kk in range(3):
      pltpu.async_copy(srcw_hbm.at[wid, kk], src_v.at[kk], sem_is)
      pltpu.async_copy(dstw_hbm.at[wid, kk], dst_v.at[kk], sem_id)
    _zero_vmem(rows_v.at[0], CK, D)
    for b in range(RPT // ZR):
      pltpu.sync_copy(
          rows_v.at[0, pl.ds(0, ZR)], acc.at[pl.ds(base + b * ZR, ZR)]
      )
    plsc.subcore_barrier()

    pltpu.make_async_copy(srcw_hbm.at[wid, 0], src_v.at[0], sem_is).wait()
    pltpu.async_copy(table_hbm.at[src_v.at[0]], rows_v.at[0], sem_g)

    def body(j, carry):
      b3 = lax.rem(j, 3)
      bi = lax.rem(j, IR)
      pltpu.make_async_copy(
          table_hbm.at[src_v.at[bi]], rows_v.at[b3], sem_g
      ).wait()

      @pl.when(j >= 2)
      def _():
        pltpu.make_async_copy(
            rows_v.at[lax.rem(j + 1, 3)],
            acc.at[dst_v.at[lax.rem(j - 2, IR)]],
            sem_s,
        ).wait()

      @pl.when(j + 1 < CH)
      def _():
        bn = lax.rem(j + 1, IR)
        pltpu.make_async_copy(srcw_hbm.at[wid, j + 1], src_v.at[bn], sem_is).wait()
        pltpu.async_copy(
            table_hbm.at[src_v.at[bn]], rows_v.at[lax.rem(j + 1, 3)], sem_g
        )

      pltpu.make_async_copy(dstw_hbm.at[wid, j], dst_v.at[bi], sem_id).wait()
      pltpu.async_copy(rows_v.at[b3], acc.at[dst_v.at[bi]], sem_s, add=True)

      @pl.when(j + 3 < CH)
      def _():
        bp = lax.rem(j + 3, IR)
        pltpu.async_copy(srcw_hbm.at[wid, j + 3], src_v.at[bp], sem_is)
        pltpu.async_copy(dstw_hbm.at[wid, j + 3], dst_v.at[bp], sem_id)

      return carry

    lax.fori_loop(0, CH, body, 0)
    for t in (CH - 2, CH - 1):
      pltpu.make_async_copy(
          rows_v.at[t % 3], acc.at[dst_v.at[t % IR]], sem_s
      ).wait()
    plsc.subcore_barrier()
    pltpu.sync_copy(
        acc.at[pl.ds(base, RPT)], out_hbm.at[cid, pl.ds(base, RPT)]
    )

  return k


_RB = 1000
_GRID = N // _RB


def _dinv_block(d_ref):
  deg = d_ref[0, :, :1] + d_ref[1, :, :1] + 1.0
  return lax.rsqrt(deg)


def _mm1_body(x_ref, w_ref, d_ref, o_ref):
  o_ref[...] = jnp.dot(
      x_ref[...], w_ref[...], preferred_element_type=jnp.float32
  ) * _dinv_block(d_ref)


def _mid_body(p_ref, g1_ref, d_ref, b1_ref, w2_ref, o_ref):
  dinv = _dinv_block(d_ref)
  s = p_ref[0] + p_ref[1] + g1_ref[...]
  t = jnp.maximum(s * dinv + b1_ref[...], 0.0)
  o_ref[...] = (
      jnp.dot(t, w2_ref[...], preferred_element_type=jnp.float32) * dinv
  )


def _final_body(p_ref, g2_ref, d_ref, b2_ref, o_ref):
  dinv = _dinv_block(d_ref)
  s = p_ref[0] + p_ref[1] + g2_ref[...]
  o = s[:, :D_OUT] * dinv + b2_ref[...]
  m = jnp.max(o, axis=1, keepdims=True)
  lse = jnp.log(jnp.sum(jnp.exp(o - m), axis=1, keepdims=True)) + m
  o_ref[...] = o - lse


def _row_spec(w):
  return pl.BlockSpec((_RB, w), lambda i: (i, 0))


def _full_spec(shape):
  nd = len(shape)
  return pl.BlockSpec(shape, lambda i: (0,) * nd)


def _deg_spec():
  return pl.BlockSpec((NC, _RB, DEG_W), lambda i: (0, i, 0))


def _part_spec(w):
  return pl.BlockSpec((NC, _RB, w), lambda i: (0, i, 0))


def kernel(x, edge_index, W1, b1, W2, b2):
  src = edge_index[0]
  dst = edge_index[1]
  pad = EP - E
  ar = jnp.arange(pad, dtype=jnp.int32)
  srcw = jnp.concatenate([src, ar % N]).reshape(NW, CH, CK)
  dstw = jnp.concatenate([dst, N + ar % CK]).reshape(NW, CH, CK)

  cz = jnp.stack([
      jnp.zeros((CK, DEG_W), jnp.float32),
      jnp.ones((CK, DEG_W), jnp.float32),
  ])
  degp = _make_sc_degree()(dstw, cz)

  g1 = pl.pallas_call(
      _mm1_body,
      grid=(_GRID,),
      in_specs=[_row_spec(D_IN), _full_spec((D_IN, D_H)), _deg_spec()],
      out_specs=_row_spec(D_H),
      out_shape=jax.ShapeDtypeStruct((N, D_H), jnp.float32),
  )(x, W1, degp)

  p1 = _make_sc_scatter(D_H)(g1, srcw, dstw)

  b1r = b1.reshape(1, D_H)
  w2p = jnp.zeros((D_H, D2P), jnp.float32).at[:, :D_OUT].set(W2)
  g2 = pl.pallas_call(
      _mid_body,
      grid=(_GRID,),
      in_specs=[
          _part_spec(D_H),
          _row_spec(D_H),
          _deg_spec(),
          _full_spec((1, D_H)),
          _full_spec((D_H, D2P)),
      ],
      out_specs=_row_spec(D2P),
      out_shape=jax.ShapeDtypeStruct((N, D2P), jnp.float32),
  )(p1, g1, degp, b1r, w2p)

  p2 = _make_sc_scatter(D2P)(g2, srcw, dstw)

  b2r = b2.reshape(1, D_OUT)
  out = pl.pallas_call(
      _final_body,
      grid=(_GRID,),
      in_specs=[
          _part_spec(D2P),
          _row_spec(D2P),
          _deg_spec(),
          _full_spec((1, D_OUT)),
      ],
      out_specs=_row_spec(D_OUT),
      out_shape=jax.ShapeDtypeStruct((N, D_OUT), jnp.float32),
  )(p2, g2, degp, b2r)

  return out


# speedup vs baseline: 1.2819x; 1.0174x over previous
import functools

import jax
import jax.numpy as jnp
from jax import lax
from jax.experimental import pallas as pl
from jax.experimental.pallas import tpu as pltpu
from jax.experimental.pallas import tpu_sc as plsc

N = 10000
E = 320000
D_IN = 128
D_H = 128
D_OUT = 40
D2P = 48

NC = 2
NS = 16
NW = NC * NS
CK = 112
CH = 90
EP = NW * CH * CK
NP = 10240
RPT = NP // NS
ZR = 64
IR = 8
DEG_W = 8

def _zero_vmem(ref, rows, width):
  def zr(r, carry):
    def zc(c, carry2):
      ref[r, pl.ds(c * 16, 16)] = jnp.zeros((16,), jnp.float32)
      return carry2

    return lax.fori_loop(0, width // 16, zc, carry)

  lax.fori_loop(0, rows, zr, 0)


@functools.lru_cache(maxsize=None)
def _make_sc_degree():
  mesh = plsc.VectorSubcoreMesh(core_axis_name="c", subcore_axis_name="s")

  @functools.partial(
      pl.kernel,
      out_type=jax.ShapeDtypeStruct((NC, NP, DEG_W), jnp.float32),
      mesh=mesh,
      compiler_params=pltpu.CompilerParams(use_tc_tiling_on_sc=False),
      scratch_types=[
          pltpu.VMEM((CH, CK), jnp.int32),
          pltpu.VMEM((2, CK, DEG_W), jnp.float32),
          pltpu.VMEM_SHARED((NP, DEG_W), jnp.float32),
          pltpu.SemaphoreType.DMA,
      ],
  )
  def _sc_degree(dstw_hbm, cz_hbm, out_hbm, dst_v, rows_v, acc, sem):
    cid = lax.axis_index("c")
    sid = lax.axis_index("s")
    wid = sid * NC + cid
    base = sid * RPT

    pltpu.sync_copy(cz_hbm, rows_v)
    for b in range(RPT // ZR):
      pltpu.sync_copy(
          rows_v.at[0, pl.ds(0, ZR)], acc.at[pl.ds(base + b * ZR, ZR)]
      )

    pltpu.sync_copy(dstw_hbm.at[wid], dst_v)
    plsc.subcore_barrier()

    def body(j, carry):
      @pl.when(j >= 8)
      def _():
        pltpu.make_async_copy(rows_v.at[1], acc.at[dst_v.at[j - 8]], sem).wait()

      pltpu.async_copy(rows_v.at[1], acc.at[dst_v.at[j]], sem, add=True)
      return carry

    lax.fori_loop(0, CH, body, 0)
    for t in range(CH - 8, CH):
      pltpu.make_async_copy(rows_v.at[1], acc.at[dst_v.at[t]], sem).wait()
    plsc.subcore_barrier()
    pltpu.sync_copy(
        acc.at[pl.ds(base, RPT)], out_hbm.at[cid, pl.ds(base, RPT)]
    )

  return _sc_degree


@functools.lru_cache(maxsize=None)
def _make_sc_scatter(D):
  mesh = plsc.VectorSubcoreMesh(core_axis_name="c", subcore_axis_name="s")

  @functools.partial(
      pl.kernel,
      out_type=jax.ShapeDtypeStruct((NC, NP, D), jnp.float32),
      mesh=mesh,
      compiler_params=pltpu.CompilerParams(use_tc_tiling_on_sc=False),
      scratch_types=[
          pltpu.VMEM((IR, CK), jnp.int32),
          pltpu.VMEM((IR, CK), jnp.int32),
          pltpu.VMEM((3, CK, D), jnp.float32),
          pltpu.VMEM_SHARED((NP, D), jnp.float32),
          pltpu.SemaphoreType.DMA,
          pltpu.SemaphoreType.DMA,
          pltpu.SemaphoreType.DMA,
          pltpu.SemaphoreType.DMA,
      ],
  )
  def k(
      table_hbm, srcw_hbm, dstw_hbm, out_hbm, src_v, dst_v, rows_v, acc,
      sem_g, sem_s, sem_is, sem_id,
  ):
    cid = lax.axis_index("c")
    sid = lax.axis_index("s")
    wid = sid * NC + cid
    base = sid * RPT

    for kk in range(3):
      pltpu.async_copy(srcw_hbm.at[wid, kk], src_v.at[kk], sem_is)
      pltpu.async_copy(dstw_hbm.at[wid, kk], dst_v.at[kk], sem_id)
    _zero_vmem(rows_v.at[0], CK, D)
    for b in range(RPT // ZR):
      pltpu.async_copy(
          rows_v.at[0, pl.ds(0, ZR)], acc.at[pl.ds(base + b * ZR, ZR)], sem_s
      )
    for b in range(RPT // ZR):
      pltpu.make_async_copy(
          rows_v.at[0, pl.ds(0, ZR)], acc.at[pl.ds(base + b * ZR, ZR)], sem_s
      ).wait()
    plsc.subcore_barrier()

    pltpu.make_async_copy(srcw_hbm.at[wid, 0], src_v.at[0], sem_is).wait()
    pltpu.async_copy(table_hbm.at[src_v.at[0]], rows_v.at[0], sem_g)

    def body(j, carry):
      b3 = lax.rem(j, 3)
      bi = lax.rem(j, IR)
      pltpu.make_async_copy(
          table_hbm.at[src_v.at[bi]], rows_v.at[b3], sem_g
      ).wait()

      @pl.when(j >= 2)
      def _():
        pltpu.make_async_copy(
            rows_v.at[lax.rem(j + 1, 3)],
            acc.at[dst_v.at[lax.rem(j - 2, IR)]],
            sem_s,
        ).wait()

      @pl.when(j + 1 < CH)
      def _():
        bn = lax.rem(j + 1, IR)
        pltpu.make_async_copy(srcw_hbm.at[wid, j + 1], src_v.at[bn], sem_is).wait()
        pltpu.async_copy(
            table_hbm.at[src_v.at[bn]], rows_v.at[lax.rem(j + 1, 3)], sem_g
        )

      pltpu.make_async_copy(dstw_hbm.at[wid, j], dst_v.at[bi], sem_id).wait()
      pltpu.async_copy(rows_v.at[b3], acc.at[dst_v.at[bi]], sem_s, add=True)

      @pl.when(j + 3 < CH)
      def _():
        bp = lax.rem(j + 3, IR)
        pltpu.async_copy(srcw_hbm.at[wid, j + 3], src_v.at[bp], sem_is)
        pltpu.async_copy(dstw_hbm.at[wid, j + 3], dst_v.at[bp], sem_id)

      return carry

    lax.fori_loop(0, CH, body, 0)
    for t in (CH - 2, CH - 1):
      pltpu.make_async_copy(
          rows_v.at[t % 3], acc.at[dst_v.at[t % IR]], sem_s
      ).wait()
    plsc.subcore_barrier()
    pltpu.sync_copy(
        acc.at[pl.ds(base, RPT)], out_hbm.at[cid, pl.ds(base, RPT)]
    )

  return k


_RB = 1000
_GRID = N // _RB


def _dinv_block(d_ref):
  deg = d_ref[0, :, :1] + d_ref[1, :, :1] + 1.0
  return lax.rsqrt(deg)


def _mm1_body(x_ref, w_ref, d_ref, o_ref):
  o_ref[...] = jnp.dot(
      x_ref[...], w_ref[...], preferred_element_type=jnp.float32
  ) * _dinv_block(d_ref)


def _mid_body(p_ref, g1_ref, d_ref, b1_ref, w2_ref, o_ref):
  dinv = _dinv_block(d_ref)
  s = p_ref[0] + p_ref[1] + g1_ref[...]
  t = jnp.maximum(s * dinv + b1_ref[...], 0.0)
  o_ref[...] = (
      jnp.dot(t, w2_ref[...], preferred_element_type=jnp.float32) * dinv
  )


def _final_body(p_ref, g2_ref, d_ref, b2_ref, o_ref):
  dinv = _dinv_block(d_ref)
  s = p_ref[0] + p_ref[1] + g2_ref[...]
  o = s[:, :D_OUT] * dinv + b2_ref[...]
  m = jnp.max(o, axis=1, keepdims=True)
  lse = jnp.log(jnp.sum(jnp.exp(o - m), axis=1, keepdims=True)) + m
  o_ref[...] = o - lse


def _row_spec(w):
  return pl.BlockSpec((_RB, w), lambda i: (i, 0))


def _full_spec(shape):
  nd = len(shape)
  return pl.BlockSpec(shape, lambda i: (0,) * nd)


def _deg_spec():
  return pl.BlockSpec((NC, _RB, DEG_W), lambda i: (0, i, 0))


def _part_spec(w):
  return pl.BlockSpec((NC, _RB, w), lambda i: (0, i, 0))


def kernel(x, edge_index, W1, b1, W2, b2):
  src = edge_index[0]
  dst = edge_index[1]
  pad = EP - E
  ar = jnp.arange(pad, dtype=jnp.int32)
  srcw = jnp.concatenate([src, ar % N]).reshape(NW, CH, CK)
  dstw = jnp.concatenate([dst, N + ar % CK]).reshape(NW, CH, CK)

  cz = jnp.stack([
      jnp.zeros((CK, DEG_W), jnp.float32),
      jnp.ones((CK, DEG_W), jnp.float32),
  ])
  degp = _make_sc_degree()(dstw, cz)

  g1 = pl.pallas_call(
      _mm1_body,
      grid=(_GRID,),
      in_specs=[_row_spec(D_IN), _full_spec((D_IN, D_H)), _deg_spec()],
      out_specs=_row_spec(D_H),
      out_shape=jax.ShapeDtypeStruct((N, D_H), jnp.float32),
  )(x, W1, degp)

  p1 = _make_sc_scatter(D_H)(g1, srcw, dstw)

  b1r = b1.reshape(1, D_H)
  w2p = jnp.zeros((D_H, D2P), jnp.float32).at[:, :D_OUT].set(W2)
  g2 = pl.pallas_call(
      _mid_body,
      grid=(_GRID,),
      in_specs=[
          _part_spec(D_H),
          _row_spec(D_H),
          _deg_spec(),
          _full_spec((1, D_H)),
          _full_spec((D_H, D2P)),
      ],
      out_specs=_row_spec(D2P),
      out_shape=jax.ShapeDtypeStruct((N, D2P), jnp.float32),
  )(p1, g1, degp, b1r, w2p)

  p2 = _make_sc_scatter(D2P)(g2, srcw, dstw)

  b2r = b2.reshape(1, D_OUT)
  out = pl.pallas_call(
      _final_body,
      grid=(_GRID,),
      in_specs=[
          _part_spec(D2P),
          _row_spec(D2P),
          _deg_spec(),
          _full_spec((1, D_OUT)),
      ],
      out_specs=_row_spec(D_OUT),
      out_shape=jax.ShapeDtypeStruct((N, D_OUT), jnp.float32),
  )(p2, g2, degp, b2r)

  return out


# speedup vs baseline: 1.2845x; 1.0020x over previous
import functools

import jax
import jax.numpy as jnp
from jax import lax
from jax.experimental import pallas as pl
from jax.experimental.pallas import tpu as pltpu
from jax.experimental.pallas import tpu_sc as plsc

N = 10000
E = 320000
D_IN = 128
D_H = 128
D_OUT = 40
D2P = 40

NC = 2
NS = 16
NW = NC * NS
CK = 112
CH = 90
EP = NW * CH * CK
NP = 10240
RPT = NP // NS
ZR = 64
IR = 8
DEG_W = 8

def _zero_vmem(ref, rows, width):
  offs = list(range(0, width - 15, 16))
  if width % 16:
    offs.append(width - 16)

  def zr(r, carry):
    for c in offs:
      ref[r, pl.ds(c, 16)] = jnp.zeros((16,), jnp.float32)
    return carry

  lax.fori_loop(0, rows, zr, 0)


@functools.lru_cache(maxsize=None)
def _make_sc_degree():
  mesh = plsc.VectorSubcoreMesh(core_axis_name="c", subcore_axis_name="s")

  @functools.partial(
      pl.kernel,
      out_type=jax.ShapeDtypeStruct((NC, NP, DEG_W), jnp.float32),
      mesh=mesh,
      compiler_params=pltpu.CompilerParams(use_tc_tiling_on_sc=False),
      scratch_types=[
          pltpu.VMEM((CH, CK), jnp.int32),
          pltpu.VMEM((2, CK, DEG_W), jnp.float32),
          pltpu.VMEM_SHARED((NP, DEG_W), jnp.float32),
          pltpu.SemaphoreType.DMA,
      ],
  )
  def _sc_degree(dstw_hbm, cz_hbm, out_hbm, dst_v, rows_v, acc, sem):
    cid = lax.axis_index("c")
    sid = lax.axis_index("s")
    wid = sid * NC + cid
    base = sid * RPT

    pltpu.sync_copy(cz_hbm, rows_v)
    for b in range(RPT // ZR):
      pltpu.sync_copy(
          rows_v.at[0, pl.ds(0, ZR)], acc.at[pl.ds(base + b * ZR, ZR)]
      )

    pltpu.sync_copy(dstw_hbm.at[wid], dst_v)
    plsc.subcore_barrier()

    def body(j, carry):
      @pl.when(j >= 8)
      def _():
        pltpu.make_async_copy(rows_v.at[1], acc.at[dst_v.at[j - 8]], sem).wait()

      pltpu.async_copy(rows_v.at[1], acc.at[dst_v.at[j]], sem, add=True)
      return carry

    lax.fori_loop(0, CH, body, 0)
    for t in range(CH - 8, CH):
      pltpu.make_async_copy(rows_v.at[1], acc.at[dst_v.at[t]], sem).wait()
    plsc.subcore_barrier()
    pltpu.sync_copy(
        acc.at[pl.ds(base, RPT)], out_hbm.at[cid, pl.ds(base, RPT)]
    )

  return _sc_degree


@functools.lru_cache(maxsize=None)
def _make_sc_scatter(D):
  mesh = plsc.VectorSubcoreMesh(core_axis_name="c", subcore_axis_name="s")

  @functools.partial(
      pl.kernel,
      out_type=jax.ShapeDtypeStruct((NC, NP, D), jnp.float32),
      mesh=mesh,
      compiler_params=pltpu.CompilerParams(use_tc_tiling_on_sc=False),
      scratch_types=[
          pltpu.VMEM((IR, CK), jnp.int32),
          pltpu.VMEM((IR, CK), jnp.int32),
          pltpu.VMEM((3, CK, D), jnp.float32),
          pltpu.VMEM_SHARED((NP, D), jnp.float32),
          pltpu.SemaphoreType.DMA,
          pltpu.SemaphoreType.DMA,
          pltpu.SemaphoreType.DMA,
          pltpu.SemaphoreType.DMA,
      ],
  )
  def k(
      table_hbm, srcw_hbm, dstw_hbm, out_hbm, src_v, dst_v, rows_v, acc,
      sem_g, sem_s, sem_is, sem_id,
  ):
    cid = lax.axis_index("c")
    sid = lax.axis_index("s")
    wid = sid * NC + cid
    base = sid * RPT

    for kk in range(3):
      pltpu.async_copy(srcw_hbm.at[wid, kk], src_v.at[kk], sem_is)
      pltpu.async_copy(dstw_hbm.at[wid, kk], dst_v.at[kk], sem_id)
    _zero_vmem(rows_v.at[0], CK, D)
    for b in range(RPT // ZR):
      pltpu.async_copy(
          rows_v.at[0, pl.ds(0, ZR)], acc.at[pl.ds(base + b * ZR, ZR)], sem_s
      )
    for b in range(RPT // ZR):
      pltpu.make_async_copy(
          rows_v.at[0, pl.ds(0, ZR)], acc.at[pl.ds(base + b * ZR, ZR)], sem_s
      ).wait()
    plsc.subcore_barrier()

    pltpu.make_async_copy(srcw_hbm.at[wid, 0], src_v.at[0], sem_is).wait()
    pltpu.async_copy(table_hbm.at[src_v.at[0]], rows_v.at[0], sem_g)

    def body(j, carry):
      b3 = lax.rem(j, 3)
      bi = lax.rem(j, IR)
      pltpu.make_async_copy(
          table_hbm.at[src_v.at[bi]], rows_v.at[b3], sem_g
      ).wait()

      @pl.when(j >= 2)
      def _():
        pltpu.make_async_copy(
            rows_v.at[lax.rem(j + 1, 3)],
            acc.at[dst_v.at[lax.rem(j - 2, IR)]],
            sem_s,
        ).wait()

      @pl.when(j + 1 < CH)
      def _():
        bn = lax.rem(j + 1, IR)
        pltpu.make_async_copy(srcw_hbm.at[wid, j + 1], src_v.at[bn], sem_is).wait()
        pltpu.async_copy(
            table_hbm.at[src_v.at[bn]], rows_v.at[lax.rem(j + 1, 3)], sem_g
        )

      pltpu.make_async_copy(dstw_hbm.at[wid, j], dst_v.at[bi], sem_id).wait()
      pltpu.async_copy(rows_v.at[b3], acc.at[dst_v.at[bi]], sem_s, add=True)

      @pl.when(j + 3 < CH)
      def _():
        bp = lax.rem(j + 3, IR)
        pltpu.async_copy(srcw_hbm.at[wid, j + 3], src_v.at[bp], sem_is)
        pltpu.async_copy(dstw_hbm.at[wid, j + 3], dst_v.at[bp], sem_id)

      return carry

    lax.fori_loop(0, CH, body, 0)
    for t in (CH - 2, CH - 1):
      pltpu.make_async_copy(
          rows_v.at[t % 3], acc.at[dst_v.at[t % IR]], sem_s
      ).wait()
    plsc.subcore_barrier()
    pltpu.sync_copy(
        acc.at[pl.ds(base, RPT)], out_hbm.at[cid, pl.ds(base, RPT)]
    )

  return k


_RB = 1000
_GRID = N // _RB


def _dinv_block(d_ref):
  deg = d_ref[0, :, :1] + d_ref[1, :, :1] + 1.0
  return lax.rsqrt(deg)


def _mm1_body(x_ref, w_ref, d_ref, o_ref):
  o_ref[...] = jnp.dot(
      x_ref[...], w_ref[...], preferred_element_type=jnp.float32
  ) * _dinv_block(d_ref)


def _mid_body(p_ref, g1_ref, d_ref, b1_ref, w2_ref, o_ref):
  dinv = _dinv_block(d_ref)
  s = p_ref[0] + p_ref[1] + g1_ref[...]
  t = jnp.maximum(s * dinv + b1_ref[...], 0.0)
  o_ref[...] = (
      jnp.dot(t, w2_ref[...], preferred_element_type=jnp.float32) * dinv
  )


def _final_body(p_ref, g2_ref, d_ref, b2_ref, o_ref):
  dinv = _dinv_block(d_ref)
  s = p_ref[0] + p_ref[1] + g2_ref[...]
  o = s[:, :D_OUT] * dinv + b2_ref[...]
  m = jnp.max(o, axis=1, keepdims=True)
  lse = jnp.log(jnp.sum(jnp.exp(o - m), axis=1, keepdims=True)) + m
  o_ref[...] = o - lse


def _row_spec(w):
  return pl.BlockSpec((_RB, w), lambda i: (i, 0))


def _full_spec(shape):
  nd = len(shape)
  return pl.BlockSpec(shape, lambda i: (0,) * nd)


def _deg_spec():
  return pl.BlockSpec((NC, _RB, DEG_W), lambda i: (0, i, 0))


def _part_spec(w):
  return pl.BlockSpec((NC, _RB, w), lambda i: (0, i, 0))


def kernel(x, edge_index, W1, b1, W2, b2):
  src = edge_index[0]
  dst = edge_index[1]
  pad = EP - E
  ar = jnp.arange(pad, dtype=jnp.int32)
  srcw = jnp.concatenate([src, ar % N]).reshape(NW, CH, CK)
  dstw = jnp.concatenate([dst, N + ar % CK]).reshape(NW, CH, CK)

  cz = jnp.stack([
      jnp.zeros((CK, DEG_W), jnp.float32),
      jnp.ones((CK, DEG_W), jnp.float32),
  ])
  degp = _make_sc_degree()(dstw, cz)

  g1 = pl.pallas_call(
      _mm1_body,
      grid=(_GRID,),
      in_specs=[_row_spec(D_IN), _full_spec((D_IN, D_H)), _deg_spec()],
      out_specs=_row_spec(D_H),
      out_shape=jax.ShapeDtypeStruct((N, D_H), jnp.float32),
  )(x, W1, degp)

  p1 = _make_sc_scatter(D_H)(g1, srcw, dstw)

  b1r = b1.reshape(1, D_H)
  w2p = jnp.zeros((D_H, D2P), jnp.float32).at[:, :D_OUT].set(W2)
  g2 = pl.pallas_call(
      _mid_body,
      grid=(_GRID,),
      in_specs=[
          _part_spec(D_H),
          _row_spec(D_H),
          _deg_spec(),
          _full_spec((1, D_H)),
          _full_spec((D_H, D2P)),
      ],
      out_specs=_row_spec(D2P),
      out_shape=jax.ShapeDtypeStruct((N, D2P), jnp.float32),
  )(p1, g1, degp, b1r, w2p)

  p2 = _make_sc_scatter(D2P)(g2, srcw, dstw)

  b2r = b2.reshape(1, D_OUT)
  out = pl.pallas_call(
      _final_body,
      grid=(_GRID,),
      in_specs=[
          _part_spec(D2P),
          _row_spec(D2P),
          _deg_spec(),
          _full_spec((1, D_OUT)),
      ],
      out_specs=_row_spec(D_OUT),
      out_shape=jax.ShapeDtypeStruct((N, D_OUT), jnp.float32),
  )(p2, g2, degp, b2r)

  return out
